# Initial kernel scaffold; baseline (speedup 1.0000x reference)
import jax
import jax.numpy as jnp
from jax.experimental import pallas as pl


def kernel(vecs, loss_mask):
    raise NotImplementedError("write your pallas kernel here")


# speedup vs baseline: 426.2475x; 426.2475x over previous
import functools

import jax
import jax.numpy as jnp
from jax import lax
from jax.experimental import pallas as pl
from jax.experimental.pallas import tpu as pltpu
from jax.experimental.pallas import tpu_sc as plsc

_N_CODE = 1024
_D_K = 64
_PE_LAM = 100000.0
_TAU = float(_D_K) ** 0.5
_EPS = 1e-6

_SC_CORES = 2
_SC_SUBCORES = 16
_NW = _SC_CORES * _SC_SUBCORES
_CH = 128

_TL = 512


def _codebook():
    pos = jnp.arange(_N_CODE, dtype=jnp.float32)
    inv = 1.0 / (_PE_LAM ** (jnp.arange(0, _D_K, 2, dtype=jnp.float32) / _D_K))
    pre = pos[:, None] * inv[None, :]
    cat = jnp.concatenate([jnp.sin(pre), jnp.cos(pre)], axis=-1)
    ms = jnp.mean(jnp.square(cat), axis=-1, keepdims=True)
    return (_TAU ** -0.5) * (cat * lax.rsqrt(ms + _EPS))


def _argmin_body(vecs_ref, cbt_ref, cbsq_ref, mask_ref, z_ref, errs_ref,
                 commit_ref):
    v = vecs_ref[...]
    scores = lax.dot_general(
        v, cbt_ref[...], (((1,), (0,)), ((), ())),
        preferred_element_type=jnp.float32)
    vsq = jnp.sum(v * v, axis=1, keepdims=True)
    diffs2 = vsq - 2.0 * scores + cbsq_ref[...]
    m = jnp.min(diffs2, axis=1, keepdims=True)
    sidx = lax.broadcasted_iota(jnp.int32, diffs2.shape, 1)
    z = jnp.min(jnp.where(diffs2 == m, sidx, _N_CODE), axis=1)
    e = jnp.maximum(m[:, 0], 0.0)
    z_ref[0, 0, :] = z
    errs_ref[0, 0, :] = e
    commit_ref[0, 0, 0] = jnp.sum(mask_ref[0, 0, :] * e)


def _shortcodes_tc(v2, cbt, cbsq, mask3, n_rows, h, lt):
    nb = n_rows // _TL
    return pl.pallas_call(
        _argmin_body,
        grid=(nb,),
        in_specs=[
            pl.BlockSpec((_TL, _D_K), lambda t: (t, 0)),
            pl.BlockSpec((_D_K, _N_CODE), lambda t: (0, 0)),
            pl.BlockSpec((1, _N_CODE), lambda t: (0, 0)),
            pl.BlockSpec((1, 1, _TL),
                         lambda t: (t // (h * lt) * lt + t % lt, 0, 0)),
        ],
        out_specs=[
            pl.BlockSpec((1, 1, _TL), lambda t: (t, 0, 0)),
            pl.BlockSpec((1, 1, _TL), lambda t: (t, 0, 0)),
            pl.BlockSpec((1, 1, 1), lambda t: (t, 0, 0),
                         memory_space=pltpu.SMEM),
        ],
        out_shape=[
            jax.ShapeDtypeStruct((nb, 1, _TL), jnp.int32),
            jax.ShapeDtypeStruct((nb, 1, _TL), jnp.float32),
            jax.ShapeDtypeStruct((nb, 1, 1), jnp.float32),
        ],
    )(v2, cbt, cbsq, mask3)


def _gather_codewords_sc(cb, zflat):
    n = zflat.shape[0]
    per_w = n // _NW
    nchunk = per_w // _CH
    idx3 = zflat.reshape(_NW, nchunk, _CH)
    mesh = plsc.VectorSubcoreMesh(core_axis_name="c", subcore_axis_name="s")

    @functools.partial(
        pl.kernel,
        mesh=mesh,
        compiler_params=pltpu.CompilerParams(use_tc_tiling_on_sc=False),
        out_type=jax.ShapeDtypeStruct((n, _D_K), jnp.float32),
        scratch_types=[
            pltpu.VMEM((nchunk, _CH), jnp.int32),
            pltpu.VMEM((_CH, _D_K), jnp.float32),
            pltpu.SemaphoreType.DMA,
        ],
    )
    def gk(table_hbm, idx_hbm, out_hbm, idx_v, rows_v, sem):
        wid = lax.axis_index("s") * _SC_CORES + lax.axis_index("c")
        base = wid * per_w
        pltpu.sync_copy(idx_hbm.at[wid], idx_v)

        def body(j, carry):
            pltpu.async_copy(table_hbm.at[idx_v.at[j]], rows_v, sem).wait()
            pltpu.sync_copy(rows_v, out_hbm.at[pl.ds(base + j * _CH, _CH)])
            return carry

        lax.fori_loop(0, nchunk, body, 0)

    return gk(cb, idx3)


def kernel(vecs, loss_mask):
    b, h, l, d = vecs.shape
    n = b * h * l
    lt = l // _TL
    cb = _codebook()
    cbt = cb.T
    cbsq = jnp.sum(jnp.square(cb), axis=-1)[None, :]
    v2 = vecs.reshape(n, d)
    mask3 = loss_mask.reshape(b * lt, 1, _TL)
    z3, errs3, commit = _shortcodes_tc(v2, cbt, cbsq, mask3, n, h, lt)
    z = z3.reshape(b, h, l)
    errs2 = errs3.reshape(b, h, l)
    l_commit = jnp.sum(commit) / (b * l)
    vecs_hat = _gather_codewords_sc(cb, z3.reshape(n)).reshape(b, h, l, d)
    l_codebook = jnp.zeros([], jnp.float32)
    return (vecs_hat, z, l_commit, l_codebook, errs2)


# speedup vs baseline: 750.8692x; 1.7616x over previous
import functools

import jax
import jax.numpy as jnp
from jax import lax
from jax.experimental import pallas as pl
from jax.experimental.pallas import tpu as pltpu
from jax.experimental.pallas import tpu_sc as plsc

_N_CODE = 1024
_D_K = 64
_PE_LAM = 100000.0
_TAU = float(_D_K) ** 0.5
_EPS = 1e-6

_SC_CORES = 2
_SC_SUBCORES = 16
_NW = _SC_CORES * _SC_SUBCORES
_CH = 128

_TL = 512
_D_AUG = 72


def _codebook():
    pos = jnp.arange(_N_CODE, dtype=jnp.float32)
    inv = 1.0 / (_PE_LAM ** (jnp.arange(0, _D_K, 2, dtype=jnp.float32) / _D_K))
    pre = pos[:, None] * inv[None, :]
    cat = jnp.concatenate([jnp.sin(pre), jnp.cos(pre)], axis=-1)
    ms = jnp.mean(jnp.square(cat), axis=-1, keepdims=True)
    return (_TAU ** -0.5) * (cat * lax.rsqrt(ms + _EPS))


def _argmin_body(vecs_ref, cba_ref, sidx_ref, ones_ref, mask_ref, z_ref,
                 errs_ref, commit_ref):
    v = vecs_ref[...]
    pad = lax.broadcasted_iota(jnp.int32, (_TL, _D_AUG - _D_K), 1)
    v_aug = jnp.concatenate(
        [v, jnp.where(pad == 0, 1.0, 0.0)], axis=1)
    nt = (((1,), (1,)), ((), ()))
    r = lax.dot_general(cba_ref[...], v_aug, nt,
                        preferred_element_type=jnp.float32)
    m = jnp.min(r, axis=0, keepdims=True)
    zf = jnp.sum(jnp.where(r == m, sidx_ref[...], 0.0), axis=0)
    vsq = lax.dot_general(ones_ref[...], v * v, nt,
                          preferred_element_type=jnp.float32)
    e = jnp.maximum(vsq[0, :] + m[0, :], 0.0)
    z_ref[0, 0, :] = zf.astype(jnp.int32)
    errs_ref[0, 0, :] = e
    commit_ref[0, 0, 0] = jnp.sum(mask_ref[0, 0, :] * e)


def _shortcodes_tc(v2, cba, sidx, ones, mask3, n_rows, h, lt):
    nb = n_rows // _TL
    return pl.pallas_call(
        _argmin_body,
        grid=(nb,),
        in_specs=[
            pl.BlockSpec((_TL, _D_K), lambda t: (t, 0)),
            pl.BlockSpec((_N_CODE, _D_AUG), lambda t: (0, 0)),
            pl.BlockSpec((_N_CODE, _TL), lambda t: (0, 0)),
            pl.BlockSpec((1, _D_K), lambda t: (0, 0)),
            pl.BlockSpec((1, 1, _TL),
                         lambda t: (t // (h * lt) * lt + t % lt, 0, 0)),
        ],
        out_specs=[
            pl.BlockSpec((1, 1, _TL), lambda t: (t, 0, 0)),
            pl.BlockSpec((1, 1, _TL), lambda t: (t, 0, 0)),
            pl.BlockSpec((1, 1, 1), lambda t: (t, 0, 0),
                         memory_space=pltpu.SMEM),
        ],
        out_shape=[
            jax.ShapeDtypeStruct((nb, 1, _TL), jnp.int32),
            jax.ShapeDtypeStruct((nb, 1, _TL), jnp.float32),
            jax.ShapeDtypeStruct((nb, 1, 1), jnp.float32),
        ],
    )(v2, cba, sidx, ones, mask3)


def _gather_codewords_sc(cb, zflat):
    n = zflat.shape[0]
    per_w = n // _NW
    nchunk = per_w // _CH
    idx3 = zflat.reshape(_NW, nchunk, _CH)
    mesh = plsc.VectorSubcoreMesh(core_axis_name="c", subcore_axis_name="s")

    @functools.partial(
        pl.kernel,
        mesh=mesh,
        compiler_params=pltpu.CompilerParams(use_tc_tiling_on_sc=False),
        out_type=jax.ShapeDtypeStruct((n, _D_K), jnp.float32),
        scratch_types=[
            pltpu.VMEM((nchunk, _CH), jnp.int32),
            pltpu.VMEM((_CH, _D_K), jnp.float32),
            pltpu.SemaphoreType.DMA,
        ],
    )
    def gk(table_hbm, idx_hbm, out_hbm, idx_v, rows_v, sem):
        wid = lax.axis_index("s") * _SC_CORES + lax.axis_index("c")
        base = wid * per_w
        pltpu.sync_copy(idx_hbm.at[wid], idx_v)

        def body(j, carry):
            pltpu.async_copy(table_hbm.at[idx_v.at[j]], rows_v, sem).wait()
            pltpu.sync_copy(rows_v, out_hbm.at[pl.ds(base + j * _CH, _CH)])
            return carry

        lax.fori_loop(0, nchunk, body, 0)

    return gk(cb, idx3)


def kernel(vecs, loss_mask):
    b, h, l, d = vecs.shape
    n = b * h * l
    lt = l // _TL
    cb = _codebook()
    cbsq = jnp.sum(jnp.square(cb), axis=-1)[:, None]
    cba = jnp.concatenate(
        [-2.0 * cb, cbsq,
         jnp.zeros((_N_CODE, _D_AUG - _D_K - 1), jnp.float32)],
        axis=1)
    sidx = jnp.broadcast_to(
        jnp.arange(_N_CODE, dtype=jnp.float32)[:, None],
        (_N_CODE, _TL))
    ones = jnp.ones((1, d), jnp.float32)
    v2 = vecs.reshape(n, d)
    mask3 = loss_mask.reshape(b * lt, 1, _TL)
    z3, errs3, commit = _shortcodes_tc(v2, cba, sidx, ones, mask3, n, h, lt)
    z = z3.reshape(b, h, l)
    errs2 = errs3.reshape(b, h, l)
    l_commit = jnp.sum(commit) / (b * l)
    vecs_hat = _gather_codewords_sc(cb, z3.reshape(n)).reshape(b, h, l, d)
    l_codebook = jnp.zeros([], jnp.float32)
    return (vecs_hat, z, l_commit, l_codebook, errs2)


# speedup vs baseline: 761.2748x; 1.0139x over previous
import functools

import jax
import jax.numpy as jnp
from jax import lax
from jax.experimental import pallas as pl
from jax.experimental.pallas import tpu as pltpu
from jax.experimental.pallas import tpu_sc as plsc

_N_CODE = 1024
_D_K = 64
_PE_LAM = 100000.0
_TAU = float(_D_K) ** 0.5
_EPS = 1e-6

_SC_CORES = 2
_SC_SUBCORES = 16
_NW = _SC_CORES * _SC_SUBCORES
_CH = 128

_TL = 512
_D_AUG = 72


def _codebook():
    pos = jnp.arange(_N_CODE, dtype=jnp.float32)
    inv = 1.0 / (_PE_LAM ** (jnp.arange(0, _D_K, 2, dtype=jnp.float32) / _D_K))
    pre = pos[:, None] * inv[None, :]
    cat = jnp.concatenate([jnp.sin(pre), jnp.cos(pre)], axis=-1)
    ms = jnp.mean(jnp.square(cat), axis=-1, keepdims=True)
    return (_TAU ** -0.5) * (cat * lax.rsqrt(ms + _EPS))


def _argmin_body(vecs_ref, cba_ref, sidx_ref, ones_ref, mask_ref, z_ref,
                 errs_ref, commit_ref):
    v = vecs_ref[...]
    pad = lax.broadcasted_iota(jnp.int32, (_TL, _D_AUG - _D_K), 1)
    v_aug = jnp.concatenate(
        [v, jnp.where(pad == 0, 1.0, 0.0)], axis=1)
    nt = (((1,), (1,)), ((), ()))
    r = lax.dot_general(cba_ref[...], v_aug, nt,
                        preferred_element_type=jnp.float32)
    m = jnp.min(r, axis=0, keepdims=True)
    zf = jnp.sum(jnp.where(r == m, sidx_ref[...], 0.0), axis=0)
    vsq = lax.dot_general(ones_ref[...], v * v, nt,
                          preferred_element_type=jnp.float32)
    e = jnp.maximum(vsq[0, :] + m[0, :], 0.0)
    z_ref[...] = zf.astype(jnp.int32)
    errs_ref[...] = e
    commit_ref[0, 0, 0] = jnp.sum(mask_ref[0, 0, :] * e)


def _shortcodes_tc(v2, cba, sidx, ones, mask3, n_rows, h, lt):
    nb = n_rows // _TL
    return pl.pallas_call(
        _argmin_body,
        grid=(nb,),
        in_specs=[
            pl.BlockSpec((_TL, _D_K), lambda t: (t, 0)),
            pl.BlockSpec((_N_CODE, _D_AUG), lambda t: (0, 0)),
            pl.BlockSpec((_N_CODE, _TL), lambda t: (0, 0)),
            pl.BlockSpec((1, _D_K), lambda t: (0, 0)),
            pl.BlockSpec((1, 1, _TL),
                         lambda t: (t // (h * lt) * lt + t % lt, 0, 0)),
        ],
        out_specs=[
            pl.BlockSpec((_TL,), lambda t: (t,)),
            pl.BlockSpec((_TL,), lambda t: (t,)),
            pl.BlockSpec((1, 1, 1), lambda t: (t, 0, 0),
                         memory_space=pltpu.SMEM),
        ],
        out_shape=[
            jax.ShapeDtypeStruct((n_rows,), jnp.int32),
            jax.ShapeDtypeStruct((n_rows,), jnp.float32),
            jax.ShapeDtypeStruct((nb, 1, 1), jnp.float32),
        ],
    )(v2, cba, sidx, ones, mask3)


def _gather_codewords_sc(cb, zflat):
    n = zflat.shape[0]
    per_w = n // _NW
    nchunk = per_w // _CH
    mesh = plsc.VectorSubcoreMesh(core_axis_name="c", subcore_axis_name="s")

    @functools.partial(
        pl.kernel,
        mesh=mesh,
        compiler_params=pltpu.CompilerParams(use_tc_tiling_on_sc=False),
        out_type=jax.ShapeDtypeStruct((n, _D_K), jnp.float32),
        scratch_types=[
            pltpu.VMEM((per_w,), jnp.int32),
            pltpu.VMEM((_CH, _D_K), jnp.float32),
            pltpu.VMEM((_CH, _D_K), jnp.float32),
            pltpu.SemaphoreType.DMA,
            pltpu.SemaphoreType.DMA,
        ],
    )
    def gk(table_hbm, idx_hbm, out_hbm, idx_v, rows0, rows1, sem0, sem1):
        wid = lax.axis_index("s") * _SC_CORES + lax.axis_index("c")
        base = wid * per_w
        pltpu.sync_copy(idx_hbm.at[pl.ds(base, per_w)], idx_v)

        def start(j, rows, sem):
            return pltpu.async_copy(
                table_hbm.at[idx_v.at[pl.ds(j * _CH, _CH)]], rows, sem)

        start(0, rows0, sem0)

        def body(jj, carry):
            j0 = 2 * jj
            j1 = j0 + 1
            start(j1, rows1, sem1)
            pltpu.make_async_copy(
                table_hbm.at[idx_v.at[pl.ds(j0 * _CH, _CH)]], rows0,
                sem0).wait()
            pltpu.sync_copy(rows0, out_hbm.at[pl.ds(base + j0 * _CH, _CH)])

            @pl.when(j1 + 1 < nchunk)
            def _():
                start(j1 + 1, rows0, sem0)

            pltpu.make_async_copy(
                table_hbm.at[idx_v.at[pl.ds(j1 * _CH, _CH)]], rows1,
                sem1).wait()
            pltpu.sync_copy(rows1, out_hbm.at[pl.ds(base + j1 * _CH, _CH)])
            return carry

        lax.fori_loop(0, nchunk // 2, body, 0)

    return gk(cb, zflat)


def kernel(vecs, loss_mask):
    b, h, l, d = vecs.shape
    n = b * h * l
    lt = l // _TL
    cb = _codebook()
    cbsq = jnp.sum(jnp.square(cb), axis=-1)[:, None]
    cba = jnp.concatenate(
        [-2.0 * cb, cbsq,
         jnp.zeros((_N_CODE, _D_AUG - _D_K - 1), jnp.float32)],
        axis=1)
    sidx = jnp.broadcast_to(
        jnp.arange(_N_CODE, dtype=jnp.float32)[:, None],
        (_N_CODE, _TL))
    ones = jnp.ones((1, d), jnp.float32)
    v2 = vecs.reshape(n, d)
    mask3 = loss_mask.reshape(b * lt, 1, _TL)
    z1, errs1, commit = _shortcodes_tc(v2, cba, sidx, ones, mask3, n, h, lt)
    z = z1.reshape(b, h, l)
    errs2 = errs1.reshape(b, h, l)
    l_commit = jnp.sum(commit) / (b * l)
    vecs_hat = _gather_codewords_sc(cb, z1).reshape(b, h, l, d)
    l_codebook = jnp.zeros([], jnp.float32)
    return (vecs_hat, z, l_commit, l_codebook, errs2)


# speedup vs baseline: 856.3382x; 1.1249x over previous
import functools

import jax
import jax.numpy as jnp
from jax import lax
from jax.experimental import pallas as pl
from jax.experimental.pallas import tpu as pltpu
from jax.experimental.pallas import tpu_sc as plsc

_N_CODE = 1024
_D_K = 64
_PE_LAM = 100000.0
_TAU = float(_D_K) ** 0.5
_EPS = 1e-6

_SC_CORES = 2
_SC_SUBCORES = 16
_NW = _SC_CORES * _SC_SUBCORES
_CH = 128

_TL = 512
_D_AUG = 65


def _codebook():
    pos = jnp.arange(_N_CODE, dtype=jnp.float32)
    inv = 1.0 / (_PE_LAM ** (jnp.arange(0, _D_K, 2, dtype=jnp.float32) / _D_K))
    pre = pos[:, None] * inv[None, :]
    cat = jnp.concatenate([jnp.sin(pre), jnp.cos(pre)], axis=-1)
    ms = jnp.mean(jnp.square(cat), axis=-1, keepdims=True)
    return (_TAU ** -0.5) * (cat * lax.rsqrt(ms + _EPS))


def _argmin_body(vecs_ref, cba_ref, sidx_ref, mask_ref, z_ref,
                 errs_ref, commit_ref):
    vt = vecs_ref[0, 0]
    v_aug = jnp.concatenate(
        [vt, jnp.ones((1, _TL), jnp.float32)], axis=0)
    r = lax.dot_general(cba_ref[...], v_aug, (((1,), (0,)), ((), ())),
                        preferred_element_type=jnp.float32)
    m = jnp.min(r, axis=0, keepdims=True)
    zf = jnp.sum(jnp.where(r == m, sidx_ref[...], 0.0), axis=0)
    vsq = jnp.sum(vt * vt, axis=0, keepdims=True)
    e = jnp.maximum(vsq[0, :] + m[0, :], 0.0)
    z_ref[...] = zf.astype(jnp.int32)
    errs_ref[...] = e
    commit_ref[0, 0, 0] = jnp.sum(mask_ref[0, 0, :] * e)


def _shortcodes_tc(v4t, cba, sidx, mask3, n_rows, h, lt):
    nb = n_rows // _TL
    return pl.pallas_call(
        _argmin_body,
        grid=(nb,),
        in_specs=[
            pl.BlockSpec((1, 1, _D_K, _TL),
                         lambda t: (t // (h * lt), (t // lt) % h, 0, t % lt)),
            pl.BlockSpec((_N_CODE, _D_AUG), lambda t: (0, 0)),
            pl.BlockSpec((_N_CODE, _TL), lambda t: (0, 0)),
            pl.BlockSpec((1, 1, _TL),
                         lambda t: (t // (h * lt) * lt + t % lt, 0, 0)),
        ],
        out_specs=[
            pl.BlockSpec((_TL,), lambda t: (t,)),
            pl.BlockSpec((_TL,), lambda t: (t,)),
            pl.BlockSpec((1, 1, 1), lambda t: (t, 0, 0),
                         memory_space=pltpu.SMEM),
        ],
        out_shape=[
            jax.ShapeDtypeStruct((n_rows,), jnp.int32),
            jax.ShapeDtypeStruct((n_rows,), jnp.float32),
            jax.ShapeDtypeStruct((nb, 1, 1), jnp.float32),
        ],
    )(v4t, cba, sidx, mask3)


def _gather_codewords_sc(cb, zflat):
    n = zflat.shape[0]
    per_w = n // _NW
    nchunk = per_w // _CH
    mesh = plsc.VectorSubcoreMesh(core_axis_name="c", subcore_axis_name="s")

    @functools.partial(
        pl.kernel,
        mesh=mesh,
        compiler_params=pltpu.CompilerParams(use_tc_tiling_on_sc=False),
        out_type=jax.ShapeDtypeStruct((n, _D_K), jnp.float32),
        scratch_types=[
            pltpu.VMEM((per_w,), jnp.int32),
            pltpu.VMEM((_CH, _D_K), jnp.float32),
            pltpu.VMEM((_CH, _D_K), jnp.float32),
            pltpu.SemaphoreType.DMA,
            pltpu.SemaphoreType.DMA,
        ],
    )
    def gk(table_hbm, idx_hbm, out_hbm, idx_v, rows0, rows1, sem0, sem1):
        wid = lax.axis_index("s") * _SC_CORES + lax.axis_index("c")
        base = wid * per_w
        pltpu.sync_copy(idx_hbm.at[pl.ds(base, per_w)], idx_v)

        def start(j, rows, sem):
            return pltpu.async_copy(
                table_hbm.at[idx_v.at[pl.ds(j * _CH, _CH)]], rows, sem)

        start(0, rows0, sem0)

        def body(jj, carry):
            j0 = 2 * jj
            j1 = j0 + 1
            start(j1, rows1, sem1)
            pltpu.make_async_copy(
                table_hbm.at[idx_v.at[pl.ds(j0 * _CH, _CH)]], rows0,
                sem0).wait()
            pltpu.sync_copy(rows0, out_hbm.at[pl.ds(base + j0 * _CH, _CH)])

            @pl.when(j1 + 1 < nchunk)
            def _():
                start(j1 + 1, rows0, sem0)

            pltpu.make_async_copy(
                table_hbm.at[idx_v.at[pl.ds(j1 * _CH, _CH)]], rows1,
                sem1).wait()
            pltpu.sync_copy(rows1, out_hbm.at[pl.ds(base + j1 * _CH, _CH)])
            return carry

        lax.fori_loop(0, nchunk // 2, body, 0)

    return gk(cb, zflat)


def kernel(vecs, loss_mask):
    b, h, l, d = vecs.shape
    n = b * h * l
    lt = l // _TL
    cb = _codebook()
    cbsq = jnp.sum(jnp.square(cb), axis=-1)[:, None]
    cba = jnp.concatenate([-2.0 * cb, cbsq], axis=1)
    sidx = jnp.broadcast_to(
        jnp.arange(_N_CODE, dtype=jnp.float32)[:, None],
        (_N_CODE, _TL))
    mask3 = loss_mask.reshape(b * lt, 1, _TL)
    v4t = jnp.transpose(vecs, (0, 1, 3, 2))
    z1, errs1, commit = _shortcodes_tc(v4t, cba, sidx, mask3, n, h, lt)
    z = z1.reshape(b, h, l)
    errs2 = errs1.reshape(b, h, l)
    l_commit = jnp.sum(commit) / (b * l)
    vecs_hat = _gather_codewords_sc(cb, z1).reshape(b, h, l, d)
    l_codebook = jnp.zeros([], jnp.float32)
    return (vecs_hat, z, l_commit, l_codebook, errs2)


# speedup vs baseline: 868.0289x; 1.0137x over previous
import functools

import jax
import jax.numpy as jnp
from jax import lax
from jax.experimental import pallas as pl
from jax.experimental.pallas import tpu as pltpu
from jax.experimental.pallas import tpu_sc as plsc

_N_CODE = 1024
_D_K = 64
_PE_LAM = 100000.0
_TAU = float(_D_K) ** 0.5
_EPS = 1e-6

_SC_CORES = 2
_SC_SUBCORES = 16
_NW = _SC_CORES * _SC_SUBCORES
_CH = 128

_TL = 512
_D_AUG = 65


def _codebook():
    pos = jnp.arange(_N_CODE, dtype=jnp.float32)
    inv = 1.0 / (_PE_LAM ** (jnp.arange(0, _D_K, 2, dtype=jnp.float32) / _D_K))
    pre = pos[:, None] * inv[None, :]
    cat = jnp.concatenate([jnp.sin(pre), jnp.cos(pre)], axis=-1)
    ms = jnp.mean(jnp.square(cat), axis=-1, keepdims=True)
    return (_TAU ** -0.5) * (cat * lax.rsqrt(ms + _EPS))


def _argmin_body(vecs_ref, cba_ref, sidx_ref, mask_ref, z_ref,
                 errs_ref, commit_ref):
    vt = vecs_ref[0, 0]
    v_aug = jnp.concatenate(
        [vt, jnp.ones((1, _TL), jnp.float32)], axis=0)
    r = lax.dot_general(cba_ref[...], v_aug, (((1,), (0,)), ((), ())),
                        preferred_element_type=jnp.float32)
    m = jnp.min(r, axis=0, keepdims=True)
    zf = jnp.sum(jnp.where(r == m, sidx_ref[...], 0.0), axis=0)
    vsq = jnp.sum(vt * vt, axis=0, keepdims=True)
    e = jnp.maximum(vsq[0, :] + m[0, :], 0.0)
    z_ref[...] = zf.astype(jnp.int32)
    errs_ref[...] = e
    commit_ref[0, 0, 0] = jnp.sum(mask_ref[0, 0, :] * e)


def _shortcodes_tc(v4t, cba, sidx, mask3, n_rows, h0, hc, lt):
    nb = n_rows // _TL
    return pl.pallas_call(
        _argmin_body,
        grid=(nb,),
        in_specs=[
            pl.BlockSpec((1, 1, _D_K, _TL),
                         lambda t: (t // (hc * lt), h0 + (t // lt) % hc, 0,
                                    t % lt)),
            pl.BlockSpec((_N_CODE, _D_AUG), lambda t: (0, 0)),
            pl.BlockSpec((_N_CODE, _TL), lambda t: (0, 0)),
            pl.BlockSpec((1, 1, _TL),
                         lambda t: (t // (hc * lt) * lt + t % lt, 0, 0)),
        ],
        out_specs=[
            pl.BlockSpec((_TL,), lambda t: (t,)),
            pl.BlockSpec((_TL,), lambda t: (t,)),
            pl.BlockSpec((1, 1, 1), lambda t: (t, 0, 0),
                         memory_space=pltpu.SMEM),
        ],
        out_shape=[
            jax.ShapeDtypeStruct((n_rows,), jnp.int32),
            jax.ShapeDtypeStruct((n_rows,), jnp.float32),
            jax.ShapeDtypeStruct((nb, 1, 1), jnp.float32),
        ],
    )(v4t, cba, sidx, mask3)


def _gather_codewords_sc(cb, zflat):
    n = zflat.shape[0]
    per_w = n // _NW
    nchunk = per_w // _CH
    mesh = plsc.VectorSubcoreMesh(core_axis_name="c", subcore_axis_name="s")

    @functools.partial(
        pl.kernel,
        mesh=mesh,
        compiler_params=pltpu.CompilerParams(use_tc_tiling_on_sc=False),
        out_type=jax.ShapeDtypeStruct((n, _D_K), jnp.float32),
        scratch_types=[
            pltpu.VMEM((per_w,), jnp.int32),
            pltpu.VMEM((_CH, _D_K), jnp.float32),
            pltpu.VMEM((_CH, _D_K), jnp.float32),
            pltpu.SemaphoreType.DMA,
            pltpu.SemaphoreType.DMA,
        ],
    )
    def gk(table_hbm, idx_hbm, out_hbm, idx_v, rows0, rows1, sem0, sem1):
        wid = lax.axis_index("s") * _SC_CORES + lax.axis_index("c")
        base = wid * per_w
        pltpu.sync_copy(idx_hbm.at[pl.ds(base, per_w)], idx_v)

        def start(j, rows, sem):
            return pltpu.async_copy(
                table_hbm.at[idx_v.at[pl.ds(j * _CH, _CH)]], rows, sem)

        start(0, rows0, sem0)

        def body(jj, carry):
            j0 = 2 * jj
            j1 = j0 + 1
            start(j1, rows1, sem1)
            pltpu.make_async_copy(
                table_hbm.at[idx_v.at[pl.ds(j0 * _CH, _CH)]], rows0,
                sem0).wait()
            pltpu.sync_copy(rows0, out_hbm.at[pl.ds(base + j0 * _CH, _CH)])

            @pl.when(j1 + 1 < nchunk)
            def _():
                start(j1 + 1, rows0, sem0)

            pltpu.make_async_copy(
                table_hbm.at[idx_v.at[pl.ds(j1 * _CH, _CH)]], rows1,
                sem1).wait()
            pltpu.sync_copy(rows1, out_hbm.at[pl.ds(base + j1 * _CH, _CH)])
            return carry

        lax.fori_loop(0, nchunk // 2, body, 0)

    return gk(cb, zflat)


def kernel(vecs, loss_mask):
    b, h, l, d = vecs.shape
    n = b * h * l
    lt = l // _TL
    cb = _codebook()
    cbsq = jnp.sum(jnp.square(cb), axis=-1)[:, None]
    cba = jnp.concatenate([-2.0 * cb, cbsq], axis=1)
    sidx = jnp.broadcast_to(
        jnp.arange(_N_CODE, dtype=jnp.float32)[:, None],
        (_N_CODE, _TL))
    mask3 = loss_mask.reshape(b * lt, 1, _TL)
    v4t = jnp.transpose(vecs, (0, 1, 3, 2))
    nchunks = 2
    hc = h // nchunks
    zs, es, cs, hats = [], [], [], []
    for i in range(nchunks):
        z1, errs1, commit = _shortcodes_tc(
            v4t, cba, sidx, mask3, n // nchunks, i * hc, hc, lt)
        zs.append(z1.reshape(b, hc, l))
        es.append(errs1.reshape(b, hc, l))
        cs.append(jnp.sum(commit))
        hats.append(_gather_codewords_sc(cb, z1).reshape(b, hc, l, d))
    z = jnp.concatenate(zs, axis=1)
    errs2 = jnp.concatenate(es, axis=1)
    l_commit = sum(cs) / (b * l)
    vecs_hat = jnp.concatenate(hats, axis=1)
    l_codebook = jnp.zeros([], jnp.float32)
    return (vecs_hat, z, l_commit, l_codebook, errs2)
